# Initial kernel scaffold; baseline (speedup 1.0000x reference)
import jax
import jax.numpy as jnp
from jax.experimental import pallas as pl


def kernel(inputs, adj, w1, b1, w2, b2):
    raise NotImplementedError("write your pallas kernel here")


# speedup vs baseline: 3.1949x; 3.1949x over previous
import jax
import jax.numpy as jnp
from jax import lax
from jax.experimental import pallas as pl
from jax.experimental.pallas import tpu as pltpu

_F32 = jnp.float32
_BF16 = jnp.bfloat16


def _rnn_kernel(lx1_ref,
                lx2_ref,
                lct_ref,
                lpt_ref,
                w1_ref,
                w2_ref,
                eye_ref,
                out_ref):
    C = out_ref.shape[1]
    N = lct_ref.shape[0]
    half = N // 2
    tbh = w2_ref.shape[0]
    S = lx1_ref.shape[1] // C

    lct = lct_ref[...]
    lpt = lpt_ref[...]
    w1t = w1_ref[...]
    w2t = w2_ref[...]

    hs = [jnp.zeros((tbh, N), _F32) for _ in range(C)]
    for t in range(S):
        for c in range(C):
            h = hs[c]
            lht = jnp.dot(h.astype(_BF16), lct, preferred_element_type=_F32)
            aug1 = jnp.concatenate([lht.astype(_BF16), lx1_ref[0, c * S + t]],
                                   axis=0)
            pre = jnp.dot(w1t, aug1, preferred_element_type=_F32)
            sig = 0.5 + 0.5 * jnp.tanh(0.5 * pre)
            sa = sig[:tbh]
            sb = sig[tbh:]
            rh = jnp.concatenate([sa[:, :half] * h[:, :half],
                                  sb[:, :half] * h[:, half:]], axis=1)
            u = jnp.concatenate([sa[:, half:], sb[:, half:]], axis=1)
            lrt = jnp.dot(rh.astype(_BF16), lpt, preferred_element_type=_F32)
            aug2 = jnp.concatenate([lrt.astype(_BF16), lx2_ref[0, c * S + t]],
                                   axis=0)
            cc = jnp.tanh(jnp.dot(w2t, aug2, preferred_element_type=_F32))
            hs[c] = u * h + (1.0 - u) * cc
    eye = eye_ref[...]
    for c in range(C):
        out_ref[0, c] = lax.dot_general(hs[c], eye, (((0,), (0,)), ((), ())),
                                        preferred_element_type=_F32)


def _whole(arr):
    nd = arr.ndim
    return pl.BlockSpec(arr.shape, lambda g: (0,) * nd)


def _normalized_laplacian(adj):
    n = adj.shape[0]
    m = adj + jnp.eye(n, dtype=adj.dtype)
    d_inv_sqrt = jnp.power(m.sum(axis=1), -0.5)
    d_inv_sqrt = jnp.where(jnp.isinf(d_inv_sqrt), 0.0, d_inv_sqrt)
    d_mat = jnp.diag(d_inv_sqrt)
    return jnp.matmul(jnp.matmul(m, d_mat).T, d_mat)


def kernel(inputs, adj, w1, b1, w2, b2):
    inputs = inputs.astype(_F32)
    B, S, N = inputs.shape
    H = w2.shape[1]
    half = N // 2
    TB = max(1, 128 // H)
    tbh = TB * H
    b_pad = ((B + TB - 1) // TB) * TB
    n_chains = b_pad // TB
    C = next(c for c in (4, 2, 1) if n_chains % c == 0)
    n_blk = n_chains // C

    lap = _normalized_laplacian(adj)
    idx = jnp.arange(N)
    perm = jnp.concatenate([jnp.arange(0, N, 2), jnp.arange(1, N, 2)])
    inv_perm = jnp.where(idx % 2 == 0, idx // 2, half + idx // 2)
    lct = lap[:, perm].T.astype(_BF16)
    lpt = lap[perm][:, perm].T.astype(_BF16)

    x_pad = jnp.pad(inputs, ((0, b_pad - B), (0, 0), (0, 0)))
    lx = jnp.einsum('nm,bsm->bsn', lap, x_pad,
                    precision=lax.Precision.HIGHEST)
    lxp = lx[:, :, perm]

    def pack(a):
        return a.reshape(n_chains, TB, S, N).transpose(0, 2, 1, 3)

    ones = jnp.ones((n_chains, S, 1, N), _F32)
    lx1 = jnp.concatenate([pack(lx), ones], axis=2).astype(_BF16)
    lx2 = jnp.concatenate([pack(lxp), ones], axis=2).astype(_BF16)
    lx1 = lx1.reshape(n_blk, C * S, TB + 1, N)
    lx2 = lx2.reshape(n_blk, C * S, TB + 1, N)

    eye_tb = jnp.eye(TB, dtype=_F32)
    w1h, w2h = w1[1:, :], w2[1:, :]
    w1_bd = jnp.concatenate([jnp.kron(eye_tb, w1h[:, :H]),
                             jnp.kron(eye_tb, w1h[:, H:])], axis=1)
    w2_bd = jnp.kron(eye_tb, w2h)
    e1 = jnp.concatenate(
        [jnp.concatenate([jnp.kron(eye_tb, w1[0:1, :H]),
                          jnp.kron(eye_tb, w1[0:1, H:])], axis=1),
         jnp.concatenate([jnp.tile(b1[:, :H], (1, TB)),
                          jnp.tile(b1[:, H:], (1, TB))], axis=1)], axis=0)
    e2 = jnp.concatenate([jnp.kron(eye_tb, w2[0:1, :]),
                          jnp.tile(b2, (1, TB))], axis=0)
    w1t = jnp.concatenate([w1_bd.T, e1.T], axis=1).astype(_BF16)
    w2t = jnp.concatenate([w2_bd.T, e2.T], axis=1).astype(_BF16)
    eye_h = jnp.eye(tbh, dtype=_F32)

    consts = (lct, lpt, w1t, w2t, eye_h)
    out = pl.pallas_call(
        _rnn_kernel,
        out_shape=jax.ShapeDtypeStruct((n_blk, C, N, tbh), _F32),
        grid=(n_blk,),
        in_specs=[pl.BlockSpec((1, C * S, TB + 1, N), lambda g: (g, 0, 0, 0)),
                  pl.BlockSpec((1, C * S, TB + 1, N), lambda g: (g, 0, 0, 0))]
                 + [_whole(a) for a in consts],
        out_specs=pl.BlockSpec((1, C, N, tbh), lambda g: (g, 0, 0, 0)),
        compiler_params=pltpu.CompilerParams(
            dimension_semantics=("parallel",)),
    )(lx1, lx2, *consts)

    out = out.reshape(n_chains, N, TB, H).transpose(0, 2, 1, 3)
    out = out.reshape(b_pad, N, H)
    return out[:, inv_perm, :][:B]


# speedup vs baseline: 4.0704x; 1.2740x over previous
import jax
import jax.numpy as jnp
from jax import lax
from jax.experimental import pallas as pl
from jax.experimental.pallas import tpu as pltpu

_F32 = jnp.float32
_BF16 = jnp.bfloat16


def _rnn_kernel(lx1_ref,
                lx2_ref,
                lct_ref,
                lpt_ref,
                w1_ref,
                w2_ref,
                eye_ref,
                out_ref):
    G = out_ref.shape[1]
    N = lct_ref.shape[0]
    half = N // 2
    tbh = w2_ref.shape[0]
    F = out_ref.shape[3] // tbh
    S = lx1_ref.shape[1] // G

    lct = lct_ref[...]
    lpt = lpt_ref[...]
    w1t = w1_ref[...]
    w2t = w2_ref[...]

    hs = [jnp.zeros((F * tbh, N), _F32) for _ in range(G)]
    for t in range(S):
        for g in range(G):
            h = hs[g]
            lh = jnp.dot(h.astype(_BF16), lct, preferred_element_type=_F32)
            lhb = lh.astype(_BF16)
            wide = jnp.concatenate([lhb[k * tbh:(k + 1) * tbh]
                                    for k in range(F)], axis=1)
            aug1 = jnp.concatenate([wide, lx1_ref[0, g * S + t]], axis=0)
            pre = jnp.dot(w1t, aug1, preferred_element_type=_F32)
            sig = 0.5 + 0.5 * jnp.tanh(pre)
            rh_rows, us = [], []
            for k in range(F):
                s = sig[:, k * N:(k + 1) * N]
                sa, sb = s[:tbh], s[tbh:]
                hk = h[k * tbh:(k + 1) * tbh]
                rh_rows.append(jnp.concatenate([sa[:, :half] * hk[:, :half],
                                                sb[:, :half] * hk[:, half:]],
                                               axis=1))
                us.append(jnp.concatenate([sa[:, half:], sb[:, half:]], axis=1))
            rh = jnp.concatenate(rh_rows, axis=0)
            lr = jnp.dot(rh.astype(_BF16), lpt, preferred_element_type=_F32)
            lrb = lr.astype(_BF16)
            aug2 = jnp.concatenate(
                [jnp.concatenate([lrb[k * tbh:(k + 1) * tbh]
                                  for k in range(F)], axis=1),
                 lx2_ref[0, g * S + t]], axis=0)
            cc = jnp.tanh(jnp.dot(w2t, aug2, preferred_element_type=_F32))
            hs[g] = jnp.concatenate(
                [cc[:, k * N:(k + 1) * N] +
                 us[k] * (h[k * tbh:(k + 1) * tbh] - cc[:, k * N:(k + 1) * N])
                 for k in range(F)], axis=0)
    eye = eye_ref[...]
    for g in range(G):
        out_ref[0, g] = lax.dot_general(hs[g], eye, (((0,), (0,)), ((), ())),
                                        preferred_element_type=_F32)


def _whole(arr):
    nd = arr.ndim
    return pl.BlockSpec(arr.shape, lambda i: (0,) * nd)


def _normalized_laplacian(adj):
    n = adj.shape[0]
    m = adj + jnp.eye(n, dtype=adj.dtype)
    d_inv_sqrt = jnp.power(m.sum(axis=1), -0.5)
    d_inv_sqrt = jnp.where(jnp.isinf(d_inv_sqrt), 0.0, d_inv_sqrt)
    d_mat = jnp.diag(d_inv_sqrt)
    return jnp.matmul(jnp.matmul(m, d_mat).T, d_mat)


def kernel(inputs, adj, w1, b1, w2, b2):
    inputs = inputs.astype(_F32)
    B, S, N = inputs.shape
    H = w2.shape[1]
    half = N // 2
    TB = max(1, 128 // H)
    tbh = TB * H
    b_pad = ((B + TB - 1) // TB) * TB
    n_chains = b_pad // TB
    F = next(f for f in (2, 1) if n_chains % f == 0)
    G = next(g for g in (2, 1) if (n_chains // F) % g == 0)
    n_blk = n_chains // (F * G)

    lap = _normalized_laplacian(adj)
    idx = jnp.arange(N)
    perm = jnp.concatenate([jnp.arange(0, N, 2), jnp.arange(1, N, 2)])
    inv_perm = jnp.where(idx % 2 == 0, idx // 2, half + idx // 2)
    lct = lap[:, perm].T.astype(_BF16)
    lpt = lap[perm][:, perm].T.astype(_BF16)

    x_pad = jnp.pad(inputs, ((0, b_pad - B), (0, 0), (0, 0)))
    lx = jnp.einsum('nm,bsm->bsn', lap, x_pad,
                    precision=lax.Precision.HIGHEST)
    lxp = lx[:, :, perm]

    def pack(a):
        a = a.reshape(n_chains, TB, S, N).transpose(0, 2, 1, 3)
        ones = jnp.ones((n_chains, S, 1, N), _F32)
        a = jnp.concatenate([a, ones], axis=2)
        a = a.reshape(n_blk, G, F, S, TB + 1, N).transpose(0, 1, 3, 4, 2, 5)
        return a.reshape(n_blk, G * S, TB + 1, F * N).astype(_BF16)

    lx1 = pack(lx)
    lx2 = pack(lxp)

    eye_tb = jnp.eye(TB, dtype=_F32)
    w1h, w2h = w1[1:, :], w2[1:, :]
    w1_bd = jnp.concatenate([jnp.kron(eye_tb, w1h[:, :H]),
                             jnp.kron(eye_tb, w1h[:, H:])], axis=1)
    w2_bd = jnp.kron(eye_tb, w2h)
    e1 = jnp.concatenate(
        [jnp.concatenate([jnp.kron(eye_tb, w1[0:1, :H]),
                          jnp.kron(eye_tb, w1[0:1, H:])], axis=1),
         jnp.concatenate([jnp.tile(b1[:, :H], (1, TB)),
                          jnp.tile(b1[:, H:], (1, TB))], axis=1)], axis=0)
    e2 = jnp.concatenate([jnp.kron(eye_tb, w2[0:1, :]),
                          jnp.tile(b2, (1, TB))], axis=0)
    w1t = (0.5 * jnp.concatenate([w1_bd.T, e1.T], axis=1)).astype(_BF16)
    w2t = jnp.concatenate([w2_bd.T, e2.T], axis=1).astype(_BF16)
    eye_h = jnp.eye(F * tbh, dtype=_F32)

    consts = (lct, lpt, w1t, w2t, eye_h)
    out = pl.pallas_call(
        _rnn_kernel,
        out_shape=jax.ShapeDtypeStruct((n_blk, G, N, F * tbh), _F32),
        grid=(n_blk,),
        in_specs=[pl.BlockSpec((1, G * S, TB + 1, F * N), lambda i: (i, 0, 0, 0)),
                  pl.BlockSpec((1, G * S, TB + 1, F * N), lambda i: (i, 0, 0, 0))]
                 + [_whole(a) for a in consts],
        out_specs=pl.BlockSpec((1, G, N, F * tbh), lambda i: (i, 0, 0, 0)),
        compiler_params=pltpu.CompilerParams(
            dimension_semantics=("parallel",)),
    )(lx1, lx2, *consts)

    out = out.reshape(n_blk, G, N, F, tbh).transpose(0, 1, 3, 2, 4)
    out = out.reshape(n_chains, N, TB, H).transpose(0, 2, 1, 3)
    out = out.reshape(b_pad, N, H)
    return out[:, inv_perm, :][:B]


# speedup vs baseline: 4.2960x; 1.0554x over previous
import jax
import jax.numpy as jnp
from jax import lax
from jax.experimental import pallas as pl
from jax.experimental.pallas import tpu as pltpu

_F32 = jnp.float32
_BF16 = jnp.bfloat16


def _rnn_kernel(lx1_ref,
                lx2_ref,
                lct_ref,
                lpt_ref,
                w1_ref,
                w2_ref,
                pmat_ref,
                out_ref):
    N = lct_ref.shape[0]
    half = N // 2
    tbh = w2_ref.shape[0]
    TB = lx1_ref.shape[2]
    F = lx1_ref.shape[3] // N
    H = tbh // TB
    G = out_ref.shape[1] // (F * TB)
    S = lx1_ref.shape[1] // G
    ones = jnp.ones((1, F * N), _BF16)

    lct = lct_ref[...]
    lpt = lpt_ref[...]
    w1t = w1_ref[...]
    w2t = w2_ref[...]

    hs = [jnp.zeros((F * tbh, N), _F32) for _ in range(G)]
    for t in range(S):
        for g in range(G):
            h = hs[g]
            lh = jnp.dot(h.astype(_BF16), lct, preferred_element_type=_F32)
            lhb = lh.astype(_BF16)
            wide = jnp.concatenate([lhb[k * tbh:(k + 1) * tbh]
                                    for k in range(F)], axis=1)
            aug1 = jnp.concatenate([wide, lx1_ref[0, g * S + t], ones], axis=0)
            pre = jnp.dot(w1t, aug1, preferred_element_type=_F32)
            sig = 0.5 + 0.5 * jnp.tanh(pre)
            rh_rows, us = [], []
            for k in range(F):
                s = sig[:, k * N:(k + 1) * N]
                sa, sb = s[:tbh], s[tbh:]
                hk = h[k * tbh:(k + 1) * tbh]
                rh_rows.append(jnp.concatenate([sa[:, :half] * hk[:, :half],
                                                sb[:, :half] * hk[:, half:]],
                                               axis=1))
                us.append(jnp.concatenate([sa[:, half:], sb[:, half:]], axis=1))
            rh = jnp.concatenate(rh_rows, axis=0)
            lr = jnp.dot(rh.astype(_BF16), lpt, preferred_element_type=_F32)
            lrb = lr.astype(_BF16)
            aug2 = jnp.concatenate(
                [jnp.concatenate([lrb[k * tbh:(k + 1) * tbh]
                                  for k in range(F)], axis=1),
                 lx2_ref[0, g * S + t], ones], axis=0)
            cc = jnp.tanh(jnp.dot(w2t, aug2, preferred_element_type=_F32))
            hs[g] = jnp.concatenate(
                [cc[:, k * N:(k + 1) * N] +
                 us[k] * (h[k * tbh:(k + 1) * tbh] - cc[:, k * N:(k + 1) * N])
                 for k in range(F)], axis=0)
    pmat = pmat_ref[...]
    for g in range(G):
        res = lax.dot_general(pmat, hs[g], (((1,), (1,)), ((), ())),
                              preferred_element_type=_F32)
        for c in range(F * TB):
            out_ref[0, g * F * TB + c] = res[:, c * H:(c + 1) * H]


def _whole(arr):
    nd = arr.ndim
    return pl.BlockSpec(arr.shape, lambda i: (0,) * nd)


def _normalized_laplacian(adj):
    n = adj.shape[0]
    m = adj + jnp.eye(n, dtype=adj.dtype)
    d_inv_sqrt = jnp.power(m.sum(axis=1), -0.5)
    d_inv_sqrt = jnp.where(jnp.isinf(d_inv_sqrt), 0.0, d_inv_sqrt)
    d_mat = jnp.diag(d_inv_sqrt)
    return jnp.matmul(jnp.matmul(m, d_mat).T, d_mat)


def kernel(inputs, adj, w1, b1, w2, b2):
    inputs = inputs.astype(_F32)
    B, S, N = inputs.shape
    H = w2.shape[1]
    half = N // 2
    TB = max(1, 128 // H)
    tbh = TB * H
    b_pad = ((B + TB - 1) // TB) * TB
    n_chains = b_pad // TB
    F = next(f for f in (2, 1) if n_chains % f == 0)
    G = next(g for g in (2, 1) if (n_chains // F) % g == 0)
    n_blk = n_chains // (F * G)

    lap = _normalized_laplacian(adj)
    idx = jnp.arange(N)
    perm = jnp.concatenate([jnp.arange(0, N, 2), jnp.arange(1, N, 2)])
    inv_perm = jnp.where(idx % 2 == 0, idx // 2, half + idx // 2)
    lct = lap[:, perm].T.astype(_BF16)
    lpt = lap[perm][:, perm].T.astype(_BF16)

    x_pad = jnp.pad(inputs, ((0, b_pad - B), (0, 0), (0, 0)))
    lx = jnp.einsum('nm,bsm->bsn', lap.astype(_BF16), x_pad.astype(_BF16),
                    preferred_element_type=_F32)
    lxp = lx[:, :, perm]

    def pack(a):
        a = a.reshape(n_chains, TB, S, N).transpose(0, 2, 1, 3)
        a = a.reshape(n_blk, G, F, S, TB, N).transpose(0, 1, 3, 4, 2, 5)
        return a.reshape(n_blk, G * S, TB, F * N).astype(_BF16)

    lx1 = pack(lx)
    lx2 = pack(lxp)

    eye_tb = jnp.eye(TB, dtype=_F32)
    w1h, w2h = w1[1:, :], w2[1:, :]
    w1_bd = jnp.concatenate([jnp.kron(eye_tb, w1h[:, :H]),
                             jnp.kron(eye_tb, w1h[:, H:])], axis=1)
    w2_bd = jnp.kron(eye_tb, w2h)
    e1 = jnp.concatenate(
        [jnp.concatenate([jnp.kron(eye_tb, w1[0:1, :H]),
                          jnp.kron(eye_tb, w1[0:1, H:])], axis=1),
         jnp.concatenate([jnp.tile(b1[:, :H], (1, TB)),
                          jnp.tile(b1[:, H:], (1, TB))], axis=1)], axis=0)
    e2 = jnp.concatenate([jnp.kron(eye_tb, w2[0:1, :]),
                          jnp.tile(b2, (1, TB))], axis=0)
    w1t = (0.5 * jnp.concatenate([w1_bd.T, e1.T], axis=1)).astype(_BF16)
    w2t = jnp.concatenate([w2_bd.T, e2.T], axis=1).astype(_BF16)
    pmat = jax.nn.one_hot(inv_perm, N, dtype=_F32)

    consts = (lct, lpt, w1t, w2t, pmat)
    out = pl.pallas_call(
        _rnn_kernel,
        out_shape=jax.ShapeDtypeStruct((n_blk, G * F * TB, N, H), _F32),
        grid=(n_blk,),
        in_specs=[pl.BlockSpec((1, G * S, TB, F * N), lambda i: (i, 0, 0, 0)),
                  pl.BlockSpec((1, G * S, TB, F * N), lambda i: (i, 0, 0, 0))]
                 + [_whole(a) for a in consts],
        out_specs=pl.BlockSpec((1, G * F * TB, N, H), lambda i: (i, 0, 0, 0)),
        compiler_params=pltpu.CompilerParams(
            dimension_semantics=("parallel",)),
    )(lx1, lx2, *consts)

    return out.reshape(b_pad, N, H)[:B]
